# Initial kernel scaffold; baseline (speedup 1.0000x reference)
import jax
import jax.numpy as jnp
from jax.experimental import pallas as pl


def kernel(value):
    raise NotImplementedError("write your pallas kernel here")


# speedup vs baseline: 3.8738x; 3.8738x over previous
import jax
import jax.numpy as jnp
from jax.experimental import pallas as pl
from jax.experimental.pallas import tpu as pltpu

BINS = 255
LOW = -20.0
HIGH = 20.0

BLK = 1024


def _twohot_block(value_ref, out_ref):
    x = value_ref[0, 0, :]
    v = jnp.clip(jnp.sign(x) * jnp.log1p(jnp.abs(x)), LOW, HIGH)
    pos = (v - LOW) / (HIGH - LOW) * (BINS - 1)
    iota = jax.lax.broadcasted_iota(jnp.int32, (BLK, BINS), 1).astype(jnp.float32)
    out_ref[...] = jnp.maximum(1.0 - jnp.abs(pos[:, None] - iota), 0.0)


def kernel(value):
    n = value.size
    nblocks = n // BLK
    flat = value.reshape(nblocks, 1, BLK)
    out = pl.pallas_call(
        _twohot_block,
        grid=(nblocks,),
        in_specs=[pl.BlockSpec((1, 1, BLK), lambda i: (i, 0, 0))],
        out_specs=pl.BlockSpec((BLK, BINS), lambda i: (i, 0)),
        out_shape=jax.ShapeDtypeStruct((n, BINS), jnp.float32),
        compiler_params=pltpu.CompilerParams(
            dimension_semantics=("arbitrary",),
        ),
    )(flat)
    return out.reshape(value.shape + (BINS,))


# speedup vs baseline: 3.8754x; 1.0004x over previous
import jax
import jax.numpy as jnp
from jax.experimental import pallas as pl
from jax.experimental.pallas import tpu as pltpu

BINS = 255
LOW = -20.0
HIGH = 20.0

BLK = 1024


def _twohot_block(value_ref, out_ref):
    x = value_ref[0, 0, :]
    v = jnp.clip(jnp.sign(x) * jnp.log1p(jnp.abs(x)), LOW, HIGH)
    pos = (v - LOW) / (HIGH - LOW) * (BINS - 1)
    iota = jax.lax.broadcasted_iota(jnp.int32, (BLK, BINS), 1).astype(jnp.float32)
    out_ref[...] = jnp.maximum(1.0 - jnp.abs(pos[:, None] - iota), 0.0)


def kernel(value):
    n = value.size
    nblocks = n // BLK
    flat = value.reshape(nblocks, 1, BLK)
    out = pl.pallas_call(
        _twohot_block,
        grid=(nblocks,),
        in_specs=[pl.BlockSpec((1, 1, BLK), lambda i: (i, 0, 0))],
        out_specs=pl.BlockSpec((BLK, BINS), lambda i: (i, 0)),
        out_shape=jax.ShapeDtypeStruct((n, BINS), jnp.float32),
        compiler_params=pltpu.CompilerParams(
            dimension_semantics=("parallel",),
        ),
    )(flat)
    return out.reshape(value.shape + (BINS,))


# speedup vs baseline: 5.1404x; 1.3264x over previous
import jax
import jax.numpy as jnp
from jax.experimental import pallas as pl
from jax.experimental.pallas import tpu as pltpu

BINS = 255
LOW = -20.0
HIGH = 20.0

BLK = 4096


def _twohot_block(value_ref, out_ref):
    x = value_ref[0, 0, :]
    v = jnp.clip(jnp.sign(x) * jnp.log1p(jnp.abs(x)), LOW, HIGH)
    pos = (v - LOW) / (HIGH - LOW) * (BINS - 1)
    iota = jax.lax.broadcasted_iota(jnp.int32, (BLK, BINS), 1).astype(jnp.float32)
    out_ref[...] = jnp.maximum(1.0 - jnp.abs(pos[:, None] - iota), 0.0)


def kernel(value):
    n = value.size
    nblocks = n // BLK
    flat = value.reshape(nblocks, 1, BLK)
    out = pl.pallas_call(
        _twohot_block,
        grid=(nblocks,),
        in_specs=[pl.BlockSpec((1, 1, BLK), lambda i: (i, 0, 0))],
        out_specs=pl.BlockSpec((BLK, BINS), lambda i: (i, 0)),
        out_shape=jax.ShapeDtypeStruct((n, BINS), jnp.float32),
        compiler_params=pltpu.CompilerParams(
            dimension_semantics=("parallel",),
        ),
    )(flat)
    return out.reshape(value.shape + (BINS,))
